# baseline (speedup 1.0000x reference)
import jax
import jax.numpy as jnp
from jax import lax
from jax.experimental import pallas as pl
from jax.experimental.pallas import tpu as pltpu

_CONV = 16
_NT = 5
_TW = 256
_BUFS = (0, 1, 0, 1, 2)


def _tile_copy(w_hbm, j, wbuf, sem):
    return pltpu.make_async_copy(
        w_hbm.at[pl.ds(j * _TW, _TW), :], wbuf, sem)


def _linear_kernel(x_ref, w_hbm, b_ref, o_ref, xbf_ref, w0, w1, w2, s0, s1, s2):
    i = pl.program_id(0)
    m, ck = x_ref.shape
    wbufs = (w0, w1, w2)
    sems = (s0, s1, s2)
    n_tiles = w_hbm.shape[0] // _TW

    @pl.when(i < _CONV)
    def _convert():
        xbf_ref[:, pl.ds(i * ck, ck)] = x_ref[...].astype(jnp.bfloat16)

    @pl.when(i == _CONV - 1)
    def _prefetch_first():
        _tile_copy(w_hbm, 0, w0, s0).start()

    @pl.when(i >= _CONV)
    def _matmul():
        base = (i - _CONV) * _NT
        bm = m // 2
        for t in range(_NT):
            buf = _BUFS[t]
            _tile_copy(w_hbm, base + t, wbufs[buf], sems[buf]).wait()
            nxt = _BUFS[(t + 1) % _NT]
            j_next = base + t + 1

            @pl.when(j_next < n_tiles)
            def _prefetch():
                _tile_copy(w_hbm, j_next, wbufs[nxt], sems[nxt]).start()

            w_bf = wbufs[buf][...].astype(jnp.bfloat16)
            for mo in (0, bm):
                acc = lax.dot_general(
                    xbf_ref[pl.ds(mo, bm), :], w_bf,
                    dimension_numbers=(((1,), (1,)), ((), ())),
                    preferred_element_type=jnp.float32,
                )
                o_ref[pl.ds(mo, bm), pl.ds(t * _TW, _TW)] = (
                    acc + b_ref[:, pl.ds(t * _TW, _TW)])


def kernel(x, W, b):
    M, K = x.shape
    N = W.shape[0]
    BN = _NT * _TW
    CK = K // _CONV

    b2 = b.reshape(1, N)

    out = pl.pallas_call(
        _linear_kernel,
        grid=(_CONV + N // BN,),
        in_specs=[
            pl.BlockSpec((M, CK), lambda i: (0, jnp.minimum(i, _CONV - 1))),
            pl.BlockSpec(memory_space=pl.ANY),
            pl.BlockSpec((1, BN), lambda i: (0, jnp.maximum(i - _CONV, 0))),
        ],
        out_specs=pl.BlockSpec((M, BN), lambda i: (0, jnp.maximum(i - _CONV, 0))),
        out_shape=jax.ShapeDtypeStruct((M, N), jnp.float32),
        scratch_shapes=[
            pltpu.VMEM((M, K), jnp.bfloat16),
            pltpu.VMEM((_TW, K), jnp.float32),
            pltpu.VMEM((_TW, K), jnp.float32),
            pltpu.VMEM((_TW, K), jnp.float32),
            pltpu.SemaphoreType.DMA,
            pltpu.SemaphoreType.DMA,
            pltpu.SemaphoreType.DMA,
        ],
        compiler_params=pltpu.CompilerParams(
            dimension_semantics=("arbitrary",),
        ),
    )(x, W, b2)
    return out


# speedup vs baseline: 1.1867x; 1.1867x over previous
import jax
import jax.numpy as jnp
from jax import lax
from jax.experimental import pallas as pl
from jax.experimental.pallas import tpu as pltpu

_CONV = 8


def _linear_kernel(x_ref, w_ref, b_ref, o_ref, xbf_ref):
    i = pl.program_id(0)
    m, ck = x_ref.shape
    bn = w_ref.shape[0]

    @pl.when(i < _CONV)
    def _convert():
        xbf_ref[:, pl.ds(i * ck, ck)] = x_ref[...].astype(jnp.bfloat16)

    @pl.when(i >= _CONV)
    def _matmul():
        w_bf = w_ref[...].astype(jnp.bfloat16)
        bm = m // 2
        for mo in (0, bm):
            acc = lax.dot_general(
                xbf_ref[pl.ds(mo, bm), :], w_bf,
                dimension_numbers=(((1,), (1,)), ((), ())),
                preferred_element_type=jnp.float32,
            )
            o_ref[pl.ds(mo, bm), :] = acc + b_ref[...]


def _pick_bn(n):
    for bn in (256, 128):
        if n % bn == 0:
            return bn
    return n


def kernel(x, W, b):
    M, K = x.shape
    N = W.shape[0]
    BN = _pick_bn(N)
    CK = K // _CONV

    b2 = b.reshape(1, N)

    out = pl.pallas_call(
        _linear_kernel,
        grid=(_CONV + N // BN,),
        in_specs=[
            pl.BlockSpec((M, CK), lambda i: (0, jnp.minimum(i, _CONV - 1))),
            pl.BlockSpec((BN, K), lambda i: (jnp.maximum(i - _CONV, 0), 0)),
            pl.BlockSpec((1, BN), lambda i: (0, jnp.maximum(i - _CONV, 0))),
        ],
        out_specs=pl.BlockSpec((M, BN), lambda i: (0, jnp.maximum(i - _CONV, 0))),
        out_shape=jax.ShapeDtypeStruct((M, N), jnp.float32),
        scratch_shapes=[pltpu.VMEM((M, K), jnp.bfloat16)],
        compiler_params=pltpu.CompilerParams(
            dimension_semantics=("arbitrary",),
        ),
    )(x, W, b2)
    return out


# speedup vs baseline: 1.1896x; 1.0024x over previous
import jax
import jax.numpy as jnp
from jax import lax
from jax.experimental import pallas as pl
from jax.experimental.pallas import tpu as pltpu

_CONV = 4


def _linear_kernel(x_ref, w_ref, b_ref, o_ref, xbf_ref):
    i = pl.program_id(0)
    m, ck = x_ref.shape
    bn = w_ref.shape[0]

    @pl.when(i < _CONV)
    def _convert():
        xbf_ref[:, pl.ds(i * ck, ck)] = x_ref[...].astype(jnp.bfloat16)

    @pl.when(i >= _CONV)
    def _matmul():
        w_bf = w_ref[...].astype(jnp.bfloat16)
        bm = m // 2
        for mo in (0, bm):
            acc = lax.dot_general(
                xbf_ref[pl.ds(mo, bm), :], w_bf,
                dimension_numbers=(((1,), (1,)), ((), ())),
                preferred_element_type=jnp.float32,
            )
            o_ref[pl.ds(mo, bm), :] = acc + b_ref[...]


def _pick_bn(n):
    for bn in (256, 128):
        if n % bn == 0:
            return bn
    return n


def kernel(x, W, b):
    M, K = x.shape
    N = W.shape[0]
    BN = _pick_bn(N)
    CK = K // _CONV

    b2 = b.reshape(1, N)

    out = pl.pallas_call(
        _linear_kernel,
        grid=(_CONV + N // BN,),
        in_specs=[
            pl.BlockSpec((M, CK), lambda i: (0, jnp.minimum(i, _CONV - 1))),
            pl.BlockSpec((BN, K), lambda i: (jnp.maximum(i - _CONV, 0), 0)),
            pl.BlockSpec((1, BN), lambda i: (0, jnp.maximum(i - _CONV, 0))),
        ],
        out_specs=pl.BlockSpec((M, BN), lambda i: (0, jnp.maximum(i - _CONV, 0))),
        out_shape=jax.ShapeDtypeStruct((M, N), jnp.float32),
        scratch_shapes=[pltpu.VMEM((M, K), jnp.bfloat16)],
        compiler_params=pltpu.CompilerParams(
            dimension_semantics=("arbitrary",),
        ),
    )(x, W, b2)
    return out


# speedup vs baseline: 1.2023x; 1.0107x over previous
import jax
import jax.numpy as jnp
from jax import lax
from jax.experimental import pallas as pl
from jax.experimental.pallas import tpu as pltpu


def _linear_kernel(x_ref, w_ref, b_ref, o_ref):
    m = x_ref.shape[0]
    bm = m // 2
    for mo in (0, bm):
        acc = lax.dot_general(
            x_ref[pl.ds(mo, bm), :], w_ref[...],
            dimension_numbers=(((1,), (1,)), ((), ())),
            preferred_element_type=jnp.float32,
            precision=lax.Precision.DEFAULT,
        )
        o_ref[pl.ds(mo, bm), :] = acc + b_ref[...]


def kernel(x, W, b):
    M, K = x.shape
    N = W.shape[0]
    BN = 256
    b2 = b.reshape(1, N)
    out = pl.pallas_call(
        _linear_kernel,
        grid=(N // BN,),
        in_specs=[
            pl.BlockSpec((M, K), lambda i: (0, 0)),
            pl.BlockSpec((BN, K), lambda i: (i, 0)),
            pl.BlockSpec((1, BN), lambda i: (0, i)),
        ],
        out_specs=pl.BlockSpec((M, BN), lambda i: (0, i)),
        out_shape=jax.ShapeDtypeStruct((M, N), jnp.float32),
        compiler_params=pltpu.CompilerParams(
            dimension_semantics=("arbitrary",),
        ),
    )(x, W, b2)
    return out


# speedup vs baseline: 1.2023x; 1.0000x over previous
import jax
import jax.numpy as jnp
from jax import lax
from jax.experimental import pallas as pl
from jax.experimental.pallas import tpu as pltpu


def _linear_kernel(x_ref, w_ref, b_ref, o_ref):
    m = x_ref.shape[0]
    bm = m // 4
    for mo in (0, bm, 2 * bm, 3 * bm):
        acc = lax.dot_general(
            x_ref[pl.ds(mo, bm), :], w_ref[...],
            dimension_numbers=(((1,), (1,)), ((), ())),
            preferred_element_type=jnp.float32,
            precision=lax.Precision.DEFAULT,
        )
        o_ref[pl.ds(mo, bm), :] = acc + b_ref[...]


def kernel(x, W, b):
    M, K = x.shape
    N = W.shape[0]
    BN = 256
    b2 = b.reshape(1, N)
    out = pl.pallas_call(
        _linear_kernel,
        grid=(N // BN,),
        in_specs=[
            pl.BlockSpec((M, K), lambda i: (0, 0)),
            pl.BlockSpec((BN, K), lambda i: (i, 0)),
            pl.BlockSpec((1, BN), lambda i: (0, i)),
        ],
        out_specs=pl.BlockSpec((M, BN), lambda i: (0, i)),
        out_shape=jax.ShapeDtypeStruct((M, N), jnp.float32),
        compiler_params=pltpu.CompilerParams(
            dimension_semantics=("arbitrary",),
        ),
    )(x, W, b2)
    return out
